# baseline (speedup 1.0000x reference)
import functools

import jax
import jax.numpy as jnp
from jax import lax
from jax.experimental import pallas as pl
from jax.experimental.pallas import tpu as pltpu
from jax.experimental.pallas import tpu_sc as plsc

VOCAB = 1000
EMBD = 128
SEQ = 50
BATCH = 1024

NUM_CORES = 2
NUM_SUBCORES = 16
LANES = 16
NW = NUM_CORES * NUM_SUBCORES
B_PER_W = BATCH // NW
VREGS_PER_ROW = EMBD // LANES


def _gather_body(tok_hbm, pos_hbm, idx_hbm, x_hbm, pos_v, idx_v, buf, sem):
    c = lax.axis_index("c")
    s = lax.axis_index("s")
    wid = s * NUM_CORES + c
    pltpu.sync_copy(pos_hbm, pos_v)

    def chunk(i, carry):
        b = wid * B_PER_W + i
        pltpu.sync_copy(idx_hbm.at[b], idx_v)
        pltpu.async_copy(tok_hbm.at[idx_v.at[0]], buf, sem).wait()

        def row(r, rc):
            for j in range(VREGS_PER_ROW):
                sl = pl.ds(j * LANES, LANES)
                buf[r, sl] = buf[r, sl] + pos_v[r, sl]
            return rc

        lax.fori_loop(0, SEQ, row, 0)
        pltpu.sync_copy(buf, x_hbm.at[b])
        return carry

    lax.fori_loop(0, B_PER_W, chunk, 0)


@jax.jit
def _gather_stage(tok_table, pos_table, idx):
    mesh = plsc.VectorSubcoreMesh(core_axis_name="c", subcore_axis_name="s")
    return pl.kernel(
        _gather_body,
        out_type=jax.ShapeDtypeStruct((BATCH, SEQ, EMBD), jnp.float32),
        mesh=mesh,
        scratch_types=[
            pltpu.VMEM((SEQ, EMBD), jnp.float32),
            pltpu.VMEM((1, SEQ), jnp.int32),
            pltpu.VMEM((SEQ, EMBD), jnp.float32),
            pltpu.SemaphoreType.DMA,
        ],
    )(tok_table, pos_table, idx.reshape(BATCH, 1, SEQ))


def _head_body(x_ref, w_ref, b_ref, o_ref):
    o_ref[...] = (
        jnp.dot(x_ref[...], w_ref[...], preferred_element_type=jnp.float32)
        + b_ref[...]
    )


ROW_BLOCK = 512


@jax.jit
def _head_stage(x, W, b):
    rows = x.shape[0]
    return pl.pallas_call(
        _head_body,
        grid=(rows // ROW_BLOCK,),
        in_specs=[
            pl.BlockSpec((ROW_BLOCK, EMBD), lambda i: (i, 0)),
            pl.BlockSpec((EMBD, VOCAB), lambda i: (0, 0)),
            pl.BlockSpec((1, VOCAB), lambda i: (0, 0)),
        ],
        out_specs=pl.BlockSpec((ROW_BLOCK, VOCAB), lambda i: (i, 0)),
        out_shape=jax.ShapeDtypeStruct((rows, VOCAB), jnp.float32),
    )(x, W, b.reshape(1, VOCAB))


def kernel(idx, tok_table, pos_table, W, b):
    idx32 = idx.astype(jnp.int32)
    x = _gather_stage(tok_table, pos_table, idx32)
    logits = _head_stage(x.reshape(BATCH * SEQ, EMBD), W, b)
    return logits.reshape(BATCH, SEQ, VOCAB)


# speedup vs baseline: 1.0214x; 1.0214x over previous
import functools

import jax
import jax.numpy as jnp
from jax import lax
from jax.experimental import pallas as pl
from jax.experimental.pallas import tpu as pltpu
from jax.experimental.pallas import tpu_sc as plsc

VOCAB = 1000
EMBD = 128
SEQ = 50
BATCH = 1024

NUM_CORES = 2
NUM_SUBCORES = 16
LANES = 16
NW = NUM_CORES * NUM_SUBCORES
B_PER_W = BATCH // NW
VREGS_PER_ROW = EMBD // LANES


BATCHES_PER_CHUNK = 8
CHUNK_ROWS = BATCHES_PER_CHUNK * SEQ
CHUNKS = (BATCH * SEQ) // CHUNK_ROWS
CHUNKS_PER_W = CHUNKS // NW


def _gather_body(tok_hbm, pos_hbm, idx_hbm, x_hbm, pos_v, idx_v, buf, sem):
    c = lax.axis_index("c")
    s = lax.axis_index("s")
    wid = s * NUM_CORES + c
    pltpu.sync_copy(pos_hbm, pos_v)

    def chunk(i, carry):
        g = wid * CHUNKS_PER_W + i
        pltpu.sync_copy(idx_hbm.at[pl.ds(g * BATCHES_PER_CHUNK, BATCHES_PER_CHUNK)], idx_v)
        copies = []
        for r8 in range(BATCHES_PER_CHUNK):
            copies.append(pltpu.async_copy(
                tok_hbm.at[idx_v.at[r8]], buf.at[pl.ds(r8 * SEQ, SEQ)], sem))
        for cp in copies:
            cp.wait()

        def row(r, rc):
            p = lax.rem(r, SEQ)
            for j in range(VREGS_PER_ROW):
                sl = pl.ds(j * LANES, LANES)
                buf[r, sl] = buf[r, sl] + pos_v[p, sl]
            return rc

        lax.fori_loop(0, CHUNK_ROWS, row, 0)
        pltpu.sync_copy(buf, x_hbm.at[pl.ds(g * CHUNK_ROWS, CHUNK_ROWS)])
        return carry

    lax.fori_loop(0, CHUNKS_PER_W, chunk, 0)


@jax.jit
def _gather_stage(tok_table, pos_table, idx):
    mesh = plsc.VectorSubcoreMesh(core_axis_name="c", subcore_axis_name="s")
    return pl.kernel(
        _gather_body,
        out_type=jax.ShapeDtypeStruct((BATCH * SEQ, EMBD), jnp.float32),
        mesh=mesh,
        scratch_types=[
            pltpu.VMEM((SEQ, EMBD), jnp.float32),
            pltpu.VMEM((BATCHES_PER_CHUNK, SEQ), jnp.int32),
            pltpu.VMEM((CHUNK_ROWS, EMBD), jnp.float32),
            pltpu.SemaphoreType.DMA,
        ],
    )(tok_table, pos_table, idx)


def _head_body(x_ref, w_ref, b_ref, o_ref):
    o_ref[...] = (
        jnp.dot(x_ref[...], w_ref[...], preferred_element_type=jnp.float32)
        + b_ref[...]
    )


ROW_BLOCK = 512


@jax.jit
def _head_stage(x, W, b):
    rows = x.shape[0]
    return pl.pallas_call(
        _head_body,
        grid=(rows // ROW_BLOCK,),
        in_specs=[
            pl.BlockSpec((ROW_BLOCK, EMBD), lambda i: (i, 0)),
            pl.BlockSpec((EMBD, VOCAB), lambda i: (0, 0)),
            pl.BlockSpec((1, VOCAB), lambda i: (0, 0)),
        ],
        out_specs=pl.BlockSpec((ROW_BLOCK, VOCAB), lambda i: (i, 0)),
        out_shape=jax.ShapeDtypeStruct((rows, VOCAB), jnp.float32),
    )(x, W, b.reshape(1, VOCAB))


def kernel(idx, tok_table, pos_table, W, b):
    idx32 = idx.astype(jnp.int32)
    x = _gather_stage(tok_table, pos_table, idx32)
    logits = _head_stage(x, W, b)
    return logits.reshape(BATCH, SEQ, VOCAB)


# speedup vs baseline: 2.7767x; 2.7186x over previous
import jax
import jax.numpy as jnp
from jax import lax
from jax.experimental import pallas as pl
from jax.experimental.pallas import tpu as pltpu
from jax.experimental.pallas import tpu_sc as plsc

VOCAB = 1000
EMBD = 128
SEQ = 50
BATCH = 1024

NUM_CORES = 2
NUM_SUBCORES = 16
LANES = 16
NW = NUM_CORES * NUM_SUBCORES
VREGS_PER_ROW = EMBD // LANES

BATCHES_PER_CHUNK = 8
CHUNK_ROWS = BATCHES_PER_CHUNK * SEQ
CHUNKS = (BATCH * SEQ) // CHUNK_ROWS
CHUNKS_PER_W = CHUNKS // NW
SCAT_ROWS = 100
SCATS_PER_CHUNK = CHUNK_ROWS // SCAT_ROWS


def _gather_body(tok_hbm, pos_hbm, idx_hbm, dst_hbm, x_hbm,
                 pos_v, idx_v, dst_v, buf, sem):
    c = lax.axis_index("c")
    s = lax.axis_index("s")
    wid = s * NUM_CORES + c
    pltpu.sync_copy(pos_hbm, pos_v)

    def chunk(i, carry):
        g = wid * CHUNKS_PER_W + i
        pltpu.sync_copy(idx_hbm.at[pl.ds(g * BATCHES_PER_CHUNK, BATCHES_PER_CHUNK)], idx_v)
        pltpu.sync_copy(dst_hbm.at[g], dst_v)
        copies = []
        for r8 in range(BATCHES_PER_CHUNK):
            copies.append(pltpu.async_copy(
                tok_hbm.at[idx_v.at[r8]], buf.at[pl.ds(r8 * SEQ, SEQ)], sem))
        for cp in copies:
            cp.wait()

        def row(r, rc):
            p = lax.rem(r, SEQ)
            for j in range(VREGS_PER_ROW):
                sl = pl.ds(j * LANES, LANES)
                buf[r, sl] = buf[r, sl] + pos_v[p, sl]
            return rc

        lax.fori_loop(0, CHUNK_ROWS, row, 0)
        scats = []
        for q in range(SCATS_PER_CHUNK):
            scats.append(pltpu.async_copy(
                buf.at[pl.ds(q * SCAT_ROWS, SCAT_ROWS)],
                x_hbm.at[dst_v.at[q]], sem))
        for cp in scats:
            cp.wait()
        return carry

    lax.fori_loop(0, CHUNKS_PER_W, chunk, 0)


@jax.jit
def _gather_stage(tok_table, pos_table, idx, dst):
    mesh = plsc.VectorSubcoreMesh(core_axis_name="c", subcore_axis_name="s")
    return pl.kernel(
        _gather_body,
        out_type=jax.ShapeDtypeStruct((SEQ * BATCH, EMBD), jnp.float32),
        mesh=mesh,
        scratch_types=[
            pltpu.VMEM((SEQ, EMBD), jnp.float32),
            pltpu.VMEM((BATCHES_PER_CHUNK, SEQ), jnp.int32),
            pltpu.VMEM((SCATS_PER_CHUNK, SCAT_ROWS), jnp.int32),
            pltpu.VMEM((CHUNK_ROWS, EMBD), jnp.float32),
            pltpu.SemaphoreType.DMA,
        ],
    )(tok_table, pos_table, idx, dst)


def _head_body(x_ref, w_ref, b_ref, o_ref):
    xt = x_ref[0].astype(jnp.bfloat16)
    w = w_ref[...].astype(jnp.bfloat16)
    o = lax.dot_general(
        w, xt, (((0,), (1,)), ((), ())),
        preferred_element_type=jnp.float32)
    o_ref[0] = o + b_ref[...]


@jax.jit
def _head_stage(x3, W, b):
    return pl.pallas_call(
        _head_body,
        grid=(SEQ,),
        in_specs=[
            pl.BlockSpec((1, BATCH, EMBD), lambda i: (i, 0, 0)),
            pl.BlockSpec((EMBD, VOCAB), lambda i: (0, 0)),
            pl.BlockSpec((VOCAB, 1), lambda i: (0, 0)),
        ],
        out_specs=pl.BlockSpec((1, VOCAB, BATCH), lambda i: (i, 0, 0)),
        out_shape=jax.ShapeDtypeStruct((SEQ, VOCAB, BATCH), jnp.float32),
    )(x3, W, b.reshape(VOCAB, 1))


def kernel(idx, tok_table, pos_table, W, b):
    idx32 = idx.astype(jnp.int32)
    r = jnp.arange(BATCH * SEQ, dtype=jnp.int32)
    dst = ((r % SEQ) * BATCH + r // SEQ).reshape(CHUNKS, SCATS_PER_CHUNK, SCAT_ROWS)
    x = _gather_stage(tok_table, pos_table, idx32, dst)
    x3 = x.reshape(SEQ, BATCH, EMBD)
    out = _head_stage(x3, W, b)
    return out.transpose(2, 0, 1)


# speedup vs baseline: 3.7399x; 1.3469x over previous
import jax
import jax.numpy as jnp
from jax import lax
from jax.experimental import pallas as pl
from jax.experimental.pallas import tpu as pltpu
from jax.experimental.pallas import tpu_sc as plsc

VOCAB = 1000
EMBD = 128
SEQ = 50
BATCH = 1024

NUM_CORES = 2
NUM_SUBCORES = 16
LANES = 16
NW = NUM_CORES * NUM_SUBCORES
VREGS_PER_ROW = EMBD // LANES

CHUNK_ROWS = 256
CHUNKS = (BATCH * SEQ) // CHUNK_ROWS
MAX_CHUNKS_PER_W = -(-CHUNKS // NW)


def _gather_body(tok_hbm, pos_hbm, idxq_hbm, x_hbm,
                 pos_v, idx_v0, idx_v1, buf0, buf1, sem):
    c = lax.axis_index("c")
    s = lax.axis_index("s")
    wid = s * NUM_CORES + c
    pltpu.sync_copy(pos_hbm, pos_v)

    idx_vs = (idx_v0, idx_v1)
    bufs = (buf0, buf1)

    def start(i):
        g = i * NW + wid
        idx_v, buf = idx_vs[i % 2], bufs[i % 2]
        pltpu.sync_copy(idxq_hbm.at[g], idx_v)
        return [pltpu.async_copy(tok_hbm.at[idx_v.at[h]],
                                 buf.at[pl.ds(h * 128, 128)], sem)
                for h in range(2)]

    def finish(i, copies):
        g = i * NW + wid
        t = g // 4
        buf = bufs[i % 2]
        for cp in copies:
            cp.wait()
        pos = [pos_v[t, pl.ds(j * LANES, LANES)] for j in range(VREGS_PER_ROW)]

        def row(r, rc):
            for j in range(VREGS_PER_ROW):
                sl = pl.ds(j * LANES, LANES)
                buf[r, sl] = buf[r, sl] + pos[j]
            return rc

        lax.fori_loop(0, CHUNK_ROWS, row, 0)
        pltpu.sync_copy(buf, x_hbm.at[g])

    full = CHUNKS // NW
    rem = CHUNKS - full * NW
    live = start(0)
    for i in range(1, full):
        nxt = start(i)
        finish(i - 1, live)
        live = nxt
    finish(full - 1, live)
    if rem:
        @pl.when(wid < rem)
        def _():
            finish(full, start(full))


@jax.jit
def _gather_stage(tok_table, pos_table, idxq):
    mesh = plsc.VectorSubcoreMesh(core_axis_name="c", subcore_axis_name="s")
    return pl.kernel(
        _gather_body,
        out_type=jax.ShapeDtypeStruct((CHUNKS, CHUNK_ROWS, EMBD), jnp.float32),
        mesh=mesh,
        scratch_types=[
            pltpu.VMEM((SEQ, EMBD), jnp.float32),
            pltpu.VMEM((2, 128), jnp.int32),
            pltpu.VMEM((2, 128), jnp.int32),
            pltpu.VMEM((CHUNK_ROWS, EMBD), jnp.float32),
            pltpu.VMEM((CHUNK_ROWS, EMBD), jnp.float32),
            pltpu.SemaphoreType.DMA,
        ],
    )(tok_table, pos_table, idxq)


def _head_body(x_ref, w_ref, b_ref, o_ref):
    xt = x_ref[0].astype(jnp.bfloat16)
    w = w_ref[...].astype(jnp.bfloat16)
    o = lax.dot_general(
        w, xt, (((0,), (1,)), ((), ())),
        preferred_element_type=jnp.float32)
    o_ref[0] = o + b_ref[...]


@jax.jit
def _head_stage(x3, W, b):
    return pl.pallas_call(
        _head_body,
        grid=(SEQ,),
        in_specs=[
            pl.BlockSpec((1, BATCH, EMBD), lambda i: (i, 0, 0)),
            pl.BlockSpec((EMBD, VOCAB), lambda i: (0, 0)),
            pl.BlockSpec((VOCAB, 1), lambda i: (0, 0)),
        ],
        out_specs=pl.BlockSpec((1, VOCAB, BATCH), lambda i: (i, 0, 0)),
        out_shape=jax.ShapeDtypeStruct((SEQ, VOCAB, BATCH), jnp.float32),
    )(x3, W, b.reshape(VOCAB, 1))


def kernel(idx, tok_table, pos_table, W, b):
    idxq = idx.astype(jnp.int32).T.reshape(CHUNKS, 2, 128)
    x = _gather_stage(tok_table, pos_table, idxq)
    x3 = x.reshape(SEQ, BATCH, EMBD)
    out = _head_stage(x3, W, b)
    return out.transpose(2, 0, 1)


# speedup vs baseline: 3.7778x; 1.0101x over previous
import jax
import jax.numpy as jnp
from jax import lax
from jax.experimental import pallas as pl
from jax.experimental.pallas import tpu as pltpu
from jax.experimental.pallas import tpu_sc as plsc

VOCAB = 1000
EMBD = 128
SEQ = 50
BATCH = 1024

NUM_CORES = 2
NUM_SUBCORES = 16
LANES = 16
NW = NUM_CORES * NUM_SUBCORES
VREGS_PER_ROW = EMBD // LANES

CHUNK_ROWS = 256
CHUNKS = (BATCH * SEQ) // CHUNK_ROWS
MAX_CHUNKS_PER_W = -(-CHUNKS // NW)


def _gather_body(g0, n_chunks, tok_hbm, pos_hbm, idxq_hbm, x_hbm,
                 pos_v, idx_v0, idx_v1, buf0, buf1, sem):
    c = lax.axis_index("c")
    s = lax.axis_index("s")
    wid = s * NUM_CORES + c
    pltpu.sync_copy(pos_hbm, pos_v)

    idx_vs = (idx_v0, idx_v1)
    bufs = (buf0, buf1)

    def start(i):
        g = i * NW + wid
        idx_v, buf = idx_vs[i % 2], bufs[i % 2]
        pltpu.sync_copy(idxq_hbm.at[g0 + g], idx_v)
        return [pltpu.async_copy(tok_hbm.at[idx_v.at[h]],
                                 buf.at[pl.ds(h * 128, 128)], sem)
                for h in range(2)]

    def finish(i, copies):
        g = i * NW + wid
        t = (g0 + g) // 4
        buf = bufs[i % 2]
        for cp in copies:
            cp.wait()
        pos = [pos_v[t, pl.ds(j * LANES, LANES)] for j in range(VREGS_PER_ROW)]

        def row(r, rc):
            for j in range(VREGS_PER_ROW):
                sl = pl.ds(j * LANES, LANES)
                buf[r, sl] = buf[r, sl] + pos[j]
            return rc

        lax.fori_loop(0, CHUNK_ROWS, row, 0)
        pltpu.sync_copy(buf, x_hbm.at[g])

    full = n_chunks // NW
    rem = n_chunks - full * NW
    live = start(0)
    for i in range(1, full):
        nxt = start(i)
        finish(i - 1, live)
        live = nxt
    finish(full - 1, live)
    if rem:
        @pl.when(wid < rem)
        def _():
            finish(full, start(full))


def _gather_stage(tok_table, pos_table, idxq, g0, n_chunks):
    import functools
    mesh = plsc.VectorSubcoreMesh(core_axis_name="c", subcore_axis_name="s")
    return pl.kernel(
        functools.partial(_gather_body, g0, n_chunks),
        out_type=jax.ShapeDtypeStruct((n_chunks, CHUNK_ROWS, EMBD), jnp.float32),
        mesh=mesh,
        scratch_types=[
            pltpu.VMEM((SEQ, EMBD), jnp.float32),
            pltpu.VMEM((2, 128), jnp.int32),
            pltpu.VMEM((2, 128), jnp.int32),
            pltpu.VMEM((CHUNK_ROWS, EMBD), jnp.float32),
            pltpu.VMEM((CHUNK_ROWS, EMBD), jnp.float32),
            pltpu.SemaphoreType.DMA,
        ],
    )(tok_table, pos_table, idxq)


def _head_body(x_ref, w_ref, b_ref, o_ref):
    xt = x_ref[0].astype(jnp.bfloat16)
    w = w_ref[...].astype(jnp.bfloat16)
    o = lax.dot_general(
        w, xt, (((0,), (1,)), ((), ())),
        preferred_element_type=jnp.float32)
    o_ref[0] = o + b_ref[...]


def _head_body_acc(x_ref, w_ref, b_ref, prev_ref, o_ref):
    del prev_ref
    _head_body(x_ref, w_ref, b_ref, o_ref)


T_SPLIT = 14


def _head_stage_a(x3, W, b):
    nt = x3.shape[0]
    return pl.pallas_call(
        _head_body,
        grid=(nt,),
        in_specs=[
            pl.BlockSpec((1, BATCH, EMBD), lambda i: (i, 0, 0)),
            pl.BlockSpec((EMBD, VOCAB), lambda i: (0, 0)),
            pl.BlockSpec((VOCAB, 1), lambda i: (0, 0)),
        ],
        out_specs=pl.BlockSpec((1, VOCAB, BATCH), lambda i: (i, 0, 0)),
        out_shape=jax.ShapeDtypeStruct((SEQ, VOCAB, BATCH), jnp.float32),
    )(x3, W, b.reshape(VOCAB, 1))


def _head_stage_b(x3, W, b, out_prev):
    nt = x3.shape[0]
    t0 = SEQ - nt
    return pl.pallas_call(
        _head_body_acc,
        grid=(nt,),
        in_specs=[
            pl.BlockSpec((1, BATCH, EMBD), lambda i: (i, 0, 0)),
            pl.BlockSpec((EMBD, VOCAB), lambda i: (0, 0)),
            pl.BlockSpec((VOCAB, 1), lambda i: (0, 0)),
            pl.BlockSpec(memory_space=pltpu.MemorySpace.HBM),
        ],
        out_specs=pl.BlockSpec((1, VOCAB, BATCH), lambda i: (i + t0, 0, 0)),
        out_shape=jax.ShapeDtypeStruct((SEQ, VOCAB, BATCH), jnp.float32),
        input_output_aliases={3: 0},
    )(x3, W, b.reshape(VOCAB, 1), out_prev)


@jax.jit
def kernel(idx, tok_table, pos_table, W, b):
    idxq = idx.astype(jnp.int32).T.reshape(CHUNKS, 2, 128)
    ca = T_SPLIT * 4
    xa = _gather_stage(tok_table, pos_table, idxq, 0, ca)
    xb = _gather_stage(tok_table, pos_table, idxq, ca, CHUNKS - ca)
    x3a = xa.reshape(T_SPLIT, BATCH, EMBD)
    x3b = xb.reshape(SEQ - T_SPLIT, BATCH, EMBD)
    out_a = _head_stage_a(x3a, W, b)
    out = _head_stage_b(x3b, W, b, out_a)
    return out.transpose(2, 0, 1)
